# Initial kernel scaffold; baseline (speedup 1.0000x reference)
import jax
import jax.numpy as jnp
from jax.experimental import pallas as pl


def kernel(g2m_efeat, grid_nfeat, mesh_nfeat, src_idx, dst_idx, edge_W1, edge_b1, edge_W2, edge_b2, edge_g, edge_beta, dst_W1, dst_b1, dst_W2, dst_b2, dst_g, dst_beta, src_W1, src_b1, src_W2, src_b2, src_g, src_beta):
    raise NotImplementedError("write your pallas kernel here")


# speedup vs baseline: 2.4149x; 2.4149x over previous
import functools

import jax
import jax.numpy as jnp
from jax import lax
from jax.experimental import pallas as pl
from jax.experimental.pallas import tpu as pltpu
from jax.experimental.pallas import tpu_sc as plsc

F32 = jnp.float32

E, NS, ND, D, H = 160000, 50000, 10000, 128, 128

NC, NSUB = 2, 16
NW = NC * NSUB

CB = 128
NBLK = E // CB
GJ = (NBLK + NW - 1) // NW
NBLK_CORE = NBLK // NC
SJ = (NBLK_CORE + NSUB - 1) // NSUB
RB = 80
NRB = ND // RB
RJ = (NRB + NSUB - 1) // NSUB


def _silu(x):
    return x * jax.nn.sigmoid(x)


def _ln(y, g, b):
    m = jnp.mean(y, axis=-1, keepdims=True)
    v = jnp.mean((y - m) ** 2, axis=-1, keepdims=True)
    return (y - m) * lax.rsqrt(v + 1e-5) * g + b


def _grid_body(x_ref, w1b_ref, sw1_ref, sb1_ref, sw2_ref, sb2_ref, sg_ref,
               sbeta_ref, gp_ref, go_ref):
    x = x_ref[...]
    gp_ref[...] = jnp.dot(x, w1b_ref[...], preferred_element_type=F32)
    h = _silu(jnp.dot(x, sw1_ref[...], preferred_element_type=F32)
              + sb1_ref[...])
    y = jnp.dot(h, sw2_ref[...], preferred_element_type=F32) + sb2_ref[...]
    go_ref[...] = x + _ln(y, sg_ref[...], sbeta_ref[...])


def _meshproj_body(x_ref, w1c_ref, mp_ref):
    mp_ref[...] = jnp.dot(x_ref[...], w1c_ref[...],
                          preferred_element_type=F32)


def _edge_body(g2m_ref, ga_ref, gb_ref, w1a_ref, b1_ref, w2_ref, b2_ref,
               g_ref, beta_ref, out_ref):
    x = (jnp.dot(g2m_ref[...], w1a_ref[...], preferred_element_type=F32)
         + ga_ref[...] + gb_ref[...] + b1_ref[...])
    h = _silu(x)
    y = jnp.dot(h, w2_ref[...], preferred_element_type=F32) + b2_ref[...]
    out_ref[...] = _ln(y, g_ref[...], beta_ref[...])


def _dst_body(pr_ref, mesh_ref, dw1a_ref, dw1b_ref, db1_ref, dw2_ref,
              db2_ref, dg_ref, dbeta_ref, out_ref):
    p = pr_ref[...]
    agg = p[0] + p[1]
    mesh = mesh_ref[...]
    h = _silu(jnp.dot(agg, dw1a_ref[...], preferred_element_type=F32)
              + jnp.dot(mesh, dw1b_ref[...], preferred_element_type=F32)
              + db1_ref[...])
    y = jnp.dot(h, dw2_ref[...], preferred_element_type=F32) + db2_ref[...]
    out_ref[...] = mesh + _ln(y, dg_ref[...], dbeta_ref[...])


def _full(shape):
    nd = len(shape)
    return pl.BlockSpec(shape, lambda i: (0,) * nd)


def _sc_gather_body(sidx_hbm, didx_hbm, gp_hbm, mp_hbm, outg_hbm, outm_hbm,
                    sidx_v, didx_v, ga_v, gb_v, sem_a, sem_b):
    c = lax.axis_index("c")
    s = lax.axis_index("s")
    wid = s * NC + c

    def body(j, carry):
        blk = j * NW + wid

        @pl.when(blk < NBLK)
        def _():
            base = blk * CB
            pltpu.sync_copy(sidx_hbm.at[pl.ds(base, CB)], sidx_v)
            pltpu.sync_copy(didx_hbm.at[pl.ds(base, CB)], didx_v)
            cpa = pltpu.async_copy(gp_hbm.at[sidx_v], ga_v, sem_a)
            cpb = pltpu.async_copy(mp_hbm.at[didx_v], gb_v, sem_b)
            cpa.wait()
            cpb.wait()
            pltpu.sync_copy(ga_v, outg_hbm.at[pl.ds(base, CB)])
            pltpu.sync_copy(gb_v, outm_hbm.at[pl.ds(base, CB)])

        return carry

    lax.fori_loop(0, GJ, body, 0)


def _sc_scatter_body(didx_hbm, ef_hbm, out_hbm, idx_v, rows_v, zbuf_v,
                     acc_sh):
    c = lax.axis_index("c")
    s = lax.axis_index("s")

    zero = jnp.zeros((16,), F32)

    def zrow(r, carry):
        for k in range(8):
            zbuf_v[r, pl.ds(k * 16, 16)] = zero
        return carry

    lax.fori_loop(0, RB, zrow, 0)

    def zcopy(q, carry):
        rblk = q * NSUB + s

        @pl.when(rblk < NRB)
        def _():
            pltpu.sync_copy(zbuf_v, acc_sh.at[pl.ds(rblk * RB, RB)])

        return carry

    lax.fori_loop(0, RJ, zcopy, 0)
    plsc.subcore_barrier()

    def body(j, carry):
        t = j * NSUB + s

        @pl.when(t < NBLK_CORE)
        def _():
            blk = c * NBLK_CORE + t
            base = blk * CB
            pltpu.sync_copy(didx_hbm.at[pl.ds(base, CB)], idx_v)
            pltpu.sync_copy(ef_hbm.at[pl.ds(base, CB)], rows_v)
            pltpu.sync_copy(rows_v, acc_sh.at[idx_v], add=True)

        return carry

    lax.fori_loop(0, SJ, body, 0)
    plsc.subcore_barrier()

    def drain(q, carry):
        rblk = q * NSUB + s

        @pl.when(rblk < NRB)
        def _():
            pltpu.sync_copy(acc_sh.at[pl.ds(rblk * RB, RB)],
                            out_hbm.at[c].at[pl.ds(rblk * RB, RB)])

        return carry

    lax.fori_loop(0, RJ, drain, 0)


def kernel(g2m_efeat, grid_nfeat, mesh_nfeat, src_idx, dst_idx,
           edge_W1, edge_b1, edge_W2, edge_b2, edge_g, edge_beta,
           dst_W1, dst_b1, dst_W2, dst_b2, dst_g, dst_beta,
           src_W1, src_b1, src_W2, src_b2, src_g, src_beta):
    W1a, W1b, W1c = edge_W1[:D], edge_W1[D:2 * D], edge_W1[2 * D:]
    dW1a, dW1b = dst_W1[:D], dst_W1[D:]
    r2 = lambda v: v.reshape(1, D)

    GB = 1000
    grid_proj, grid_out = pl.pallas_call(
        _grid_body,
        grid=(NS // GB,),
        in_specs=[pl.BlockSpec((GB, D), lambda i: (i, 0)),
                  _full((D, H)), _full((D, H)), _full((1, H)),
                  _full((H, D)), _full((1, D)), _full((1, D)),
                  _full((1, D))],
        out_specs=[pl.BlockSpec((GB, H), lambda i: (i, 0)),
                   pl.BlockSpec((GB, D), lambda i: (i, 0))],
        out_shape=[jax.ShapeDtypeStruct((NS, H), F32),
                   jax.ShapeDtypeStruct((NS, D), F32)],
    )(grid_nfeat, W1b, src_W1, r2(src_b1), src_W2, r2(src_b2),
      r2(src_g), r2(src_beta))

    MB = 1000
    mesh_proj = pl.pallas_call(
        _meshproj_body,
        grid=(ND // MB,),
        in_specs=[pl.BlockSpec((MB, D), lambda i: (i, 0)), _full((D, H))],
        out_specs=pl.BlockSpec((MB, H), lambda i: (i, 0)),
        out_shape=jax.ShapeDtypeStruct((ND, H), F32),
    )(mesh_nfeat, W1c)

    mesh = plsc.VectorSubcoreMesh(core_axis_name="c", subcore_axis_name="s",
                                  num_cores=NC, num_subcores=NSUB)
    ga, gb = pl.kernel(
        _sc_gather_body,
        out_type=[jax.ShapeDtypeStruct((E, H), F32),
                  jax.ShapeDtypeStruct((E, H), F32)],
        mesh=mesh,
        scratch_types=[pltpu.VMEM((CB,), jnp.int32),
                       pltpu.VMEM((CB,), jnp.int32),
                       pltpu.VMEM((CB, H), F32),
                       pltpu.VMEM((CB, H), F32),
                       pltpu.SemaphoreType.DMA,
                       pltpu.SemaphoreType.DMA],
    )(src_idx, dst_idx, grid_proj, mesh_proj)

    EB = 800
    efeat = pl.pallas_call(
        _edge_body,
        grid=(E // EB,),
        in_specs=[pl.BlockSpec((EB, D), lambda i: (i, 0)),
                  pl.BlockSpec((EB, H), lambda i: (i, 0)),
                  pl.BlockSpec((EB, H), lambda i: (i, 0)),
                  _full((D, H)), _full((1, H)), _full((H, D)),
                  _full((1, D)), _full((1, D)), _full((1, D))],
        out_specs=pl.BlockSpec((EB, D), lambda i: (i, 0)),
        out_shape=jax.ShapeDtypeStruct((E, D), F32),
    )(g2m_efeat, ga, gb, W1a, r2(edge_b1), edge_W2, r2(edge_b2),
      r2(edge_g), r2(edge_beta))

    partials = pl.kernel(
        _sc_scatter_body,
        out_type=jax.ShapeDtypeStruct((NC, ND, D), F32),
        mesh=mesh,
        scratch_types=[pltpu.VMEM((CB,), jnp.int32),
                       pltpu.VMEM((CB, D), F32),
                       pltpu.VMEM((RB, D), F32),
                       pltpu.VMEM_SHARED((ND, D), F32)],
    )(dst_idx, efeat)

    DBK = 1000
    mesh_out = pl.pallas_call(
        _dst_body,
        grid=(ND // DBK,),
        in_specs=[pl.BlockSpec((NC, DBK, D), lambda i: (0, i, 0)),
                  pl.BlockSpec((DBK, D), lambda i: (i, 0)),
                  _full((D, H)), _full((D, H)), _full((1, H)),
                  _full((H, D)), _full((1, D)), _full((1, D)),
                  _full((1, D))],
        out_specs=pl.BlockSpec((DBK, D), lambda i: (i, 0)),
        out_shape=jax.ShapeDtypeStruct((ND, D), F32),
    )(partials, mesh_nfeat, dW1a, dW1b, r2(dst_b1), dst_W2, r2(dst_b2),
      r2(dst_g), r2(dst_beta))

    return (grid_out, mesh_out)
